# baseline (speedup 1.0000x reference)
import functools

import jax
import jax.numpy as jnp
from jax import lax
from jax.experimental import pallas as pl
from jax.experimental.pallas import tpu as pltpu
from jax.experimental.pallas import tpu_sc as plsc

_B, _C, _H, _W = 64, 3, 512, 512
_PLANE = _H * _W
_G = _B * _C
_NC, _NS = 2, 16
_NWORK = _NC * _NS
_PER_W = _G // _NWORK


def _sc_permute(x2, perm16):
    mesh = plsc.VectorSubcoreMesh(core_axis_name="c", subcore_axis_name="s")

    @functools.partial(
        pl.kernel,
        mesh=mesh,
        out_type=jax.ShapeDtypeStruct((_G, _PLANE), jnp.float32),
        scratch_types=[
            pltpu.VMEM((32,), jnp.int32),
            pltpu.SemaphoreType.DMA,
        ],
    )
    def body(x_hbm, perm_hbm, out_hbm, perm_v, sem):
        wid = lax.axis_index("s") * _NC + lax.axis_index("c")
        pltpu.sync_copy(perm_hbm, perm_v)
        copies = []
        for j in range(_PER_W):
            g = wid * _PER_W + j
            b = g // _C
            c = g % _C
            pc = perm_v[pl.ds(c, 16)][0]
            src = b * _C + pc
            cp = pltpu.make_async_copy(x_hbm.at[src], out_hbm.at[g], sem)
            cp.start()
            copies.append(cp)
        for cp in copies:
            cp.wait()

    return body(x2, perm16)


def kernel(x, channel_order):
    x2 = x.reshape(_G, _PLANE)
    perm16 = jnp.zeros((32,), jnp.int32).at[:_C].set(channel_order)
    out = _sc_permute(x2, perm16)
    return out.reshape(_B, _C, _H, _W)
